# baseline (speedup 1.0000x reference)
import jax
import jax.numpy as jnp
from jax import lax
from jax.experimental import pallas as pl
from jax.experimental.pallas import tpu as pltpu
from jax.experimental.pallas import tpu_sc as plsc

N = 2048
E = 65536
DH = 32
NC, NS, L = 2, 16, 16
NW = NC * NS
EPW = E // NW
SUB = 512
NSUB = EPW // SUB
GPS = SUB // L
ROWS = E // SUB

_MESH = plsc.VectorSubcoreMesh(
    core_axis_name="c", subcore_axis_name="s", num_cores=NC, num_subcores=NS)


def _tc1_body(x_ref, w1_ref, asw_ref, adw_ref, wet_ref, ater_ref, eft_ref,
              h1_ref, asrc_ref, adst_ref, bound_ref, exself_ref, et_ref):
    h = jnp.dot(x_ref[...], w1_ref[...], preferred_element_type=jnp.float32)
    h1_ref[...] = h
    asrc = jnp.dot(h, asw_ref[...], preferred_element_type=jnp.float32)
    adst = jnp.dot(h, adw_ref[...], preferred_element_type=jnp.float32)
    wevT = jnp.dot(ater_ref[...], wet_ref[...],
                   preferred_element_type=jnp.float32)
    et = jnp.dot(wevT, eft_ref[...], preferred_element_type=jnp.float32)
    em = jnp.mean(eft_ref[...], axis=1, keepdims=True)
    ets = jnp.dot(wevT, em, preferred_element_type=jnp.float32)[0, 0]
    maxet = jnp.maximum(jnp.max(et), ets)
    b = jnp.max(asrc) + adst + maxet
    bound = jnp.maximum(b, 0.2 * b)
    u = asrc + adst + ets
    exself = jnp.exp(jnp.maximum(u, 0.2 * u) - bound)
    asrc_ref[...] = asrc
    adst_ref[...] = adst
    bound_ref[...] = bound
    exself_ref[...] = exself
    et_ref[...] = et


def _tc1(x, W1, asw, adw, We1_t, ate_row, ef_t):
    f32 = jnp.float32
    return pl.pallas_call(
        _tc1_body,
        out_shape=[
            jax.ShapeDtypeStruct((N, DH), f32),
            jax.ShapeDtypeStruct((N, 1), f32),
            jax.ShapeDtypeStruct((N, 1), f32),
            jax.ShapeDtypeStruct((N, 1), f32),
            jax.ShapeDtypeStruct((N, 1), f32),
            jax.ShapeDtypeStruct((1, E), f32),
        ],
    )(x, W1, asw, adw, We1_t, ate_row, ef_t)


def _tc2_body(acc_ref, den_ref, exs_ref, h1_ref, b1_ref, w2_ref, asw_ref,
              adw_ref, h2_ref, as2_ref, ad2_ref, bo2_ref, ex2_ref):
    a = acc_ref[...]
    accsum = a[0:N] + a[N:2 * N]
    d = den_ref[...]
    densum = d[0:N] + d[N:2 * N]
    exs = exs_ref[...]
    h1l = h1_ref[...]
    h1 = (accsum + exs * h1l) / (densum + exs + 1e-16) + b1_ref[...]
    h2 = jnp.dot(h1, w2_ref[...], preferred_element_type=jnp.float32)
    h2_ref[...] = h2
    as2 = jnp.dot(h2, asw_ref[...], preferred_element_type=jnp.float32)
    ad2 = jnp.dot(h2, adw_ref[...], preferred_element_type=jnp.float32)
    b = jnp.max(as2) + ad2
    bo = jnp.maximum(b, 0.2 * b)
    u = as2 + ad2
    ex2 = jnp.exp(jnp.maximum(u, 0.2 * u) - bo)
    as2_ref[...] = as2
    ad2_ref[...] = ad2
    bo2_ref[...] = bo
    ex2_ref[...] = ex2


def _tc2(acc, den, exs, h1lin, b1r, W2, asw2, adw2):
    f32 = jnp.float32
    return pl.pallas_call(
        _tc2_body,
        out_shape=[
            jax.ShapeDtypeStruct((N, DH), f32),
            jax.ShapeDtypeStruct((N, 1), f32),
            jax.ShapeDtypeStruct((N, 1), f32),
            jax.ShapeDtypeStruct((N, 1), f32),
            jax.ShapeDtypeStruct((N, 1), f32),
        ],
    )(acc, den, exs, h1lin, b1r, W2, asw2, adw2)


def _tc3_body(acc_ref, den_ref, exs_ref, h2_ref, b2_ref, wfc1_ref, bfc1_ref,
              ap_ref, bt_ref):
    a = acc_ref[...]
    accsum = a[0:N] + a[N:2 * N]
    d = den_ref[...]
    densum = d[0:N] + d[N:2 * N]
    exs = exs_ref[...]
    h2l = h2_ref[...]
    h2 = (accsum + exs * h2l) / (densum + exs + 1e-16) + b2_ref[...]
    w = wfc1_ref[...]
    ap_ref[...] = jnp.dot(h2, w[0:DH], preferred_element_type=jnp.float32) \
        + bfc1_ref[...]
    bt_ref[...] = jnp.dot(h2, w[DH:2 * DH], preferred_element_type=jnp.float32)


def _tc3(acc, den, exs, h2lin, b2r, Wfc1, bfc1r):
    f32 = jnp.float32
    return pl.pallas_call(
        _tc3_body,
        out_shape=[
            jax.ShapeDtypeStruct((N, DH), f32),
            jax.ShapeDtypeStruct((N, DH), f32),
        ],
    )(acc, den, exs, h2lin, b2r, Wfc1, bfc1r)


def _tc4_body(r_ref, wd_ref, bfc1_ref, wfc2_ref, bfc2_ref, out_ref):
    pid = pl.program_id(0)
    bfc2v = bfc2_ref[...]

    @pl.when(pid == 0)
    def _edge():
        db = bfc2v[0, 0] - bfc2v[0, 1]
        tt = jnp.dot(r_ref[...], wd_ref[...],
                     preferred_element_type=jnp.float32) + db
        y0 = 1.0 / (1.0 + jnp.exp(-tt))
        y1 = 1.0 / (1.0 + jnp.exp(tt))
        out_ref[...] = jnp.concatenate([y0, y1], axis=1)

    @pl.when(pid != 0)
    def _fill():
        z = jnp.dot(jnp.maximum(bfc1_ref[...], 0.0), wfc2_ref[...],
                    preferred_element_type=jnp.float32) + bfc2v
        ez = jnp.exp(z - jnp.max(z))
        y = ez / jnp.sum(ez)
        ones = jnp.ones((512, 128), jnp.float32)
        out_ref[...] = jnp.concatenate([y[0, 0] * ones, y[0, 1] * ones],
                                       axis=1)


def _tc4(R4, WD, bfc1r, Wfc2, bfc2r):
    cmap = lambda i: (0, 0)
    return pl.pallas_call(
        _tc4_body,
        grid=(64,),
        in_specs=[
            pl.BlockSpec((512, 4096), cmap),
            pl.BlockSpec((4096, 128), cmap),
            pl.BlockSpec((1, DH), cmap),
            pl.BlockSpec((DH, 2), cmap),
            pl.BlockSpec((1, 2), cmap),
        ],
        out_specs=pl.BlockSpec((512, 256), lambda i: (i, 0)),
        out_shape=jax.ShapeDtypeStruct((32768, 256), jnp.float32),
    )(R4, WD, bfc1r, Wfc2, bfc2r)


def _make_sc_conv(has_et):
    f32 = jnp.float32

    def body(*refs):
        if has_et:
            (asrc_h, adst_h, bound_h, h_h, srcr_h, dstr_h, etr_h,
             acc_out, den_out,
             asrc_v, adst_v, bound_v, srcb, etb, db0, db1, db2, db3,
             exrow, msub, h_sh, acc_sh, den_sh, sem) = refs
        else:
            (asrc_h, adst_h, bound_h, h_h, srcr_h, dstr_h,
             acc_out, den_out,
             asrc_v, adst_v, bound_v, srcb, db0, db1, db2, db3,
             exrow, msub, h_sh, acc_sh, den_sh, sem) = refs
            etb = None
        dstbs = (db0, db1, db2, db3)
        c = lax.axis_index("c")
        s = lax.axis_index("s")
        w = c * NS + s
        z16 = jnp.zeros((L,), f32)

        def zb(i, _):
            msub[i, pl.ds(0, L)] = z16
            msub[i, pl.ds(L, L)] = z16
            return 0
        lax.fori_loop(0, 128, zb, 0)
        for k in range(128 // L):
            exrow[pl.ds(k * L, L)] = z16
        pltpu.sync_copy(msub.at[pl.ds(0, 128)], acc_sh.at[pl.ds(s * 128, 128)])
        pltpu.sync_copy(exrow.at[pl.ds(0, 128)], den_sh.at[pl.ds(s * 128, 128)])
        pltpu.sync_copy(h_h.at[pl.ds(s * 128, 128)],
                        h_sh.at[pl.ds(s * 128, 128)])
        pltpu.sync_copy(asrc_h, asrc_v)
        pltpu.sync_copy(adst_h, adst_v)
        pltpu.sync_copy(bound_h, bound_v)
        plsc.subcore_barrier()

        for j in range(NSUB):
            row = w * NSUB + j
            dstb = dstbs[j]
            pltpu.sync_copy(srcr_h.at[row], srcb)
            pltpu.sync_copy(dstr_h.at[row], dstb)
            if has_et:
                pltpu.sync_copy(etr_h.at[row], etb)
            gat = pltpu.async_copy(h_sh.at[srcb], msub, sem)

            def grp(g, _):
                sl = pl.ds(g * L, L)
                s16 = srcb[sl]
                d16 = dstb[sl]
                vs = plsc.load_gather(asrc_v, [s16])
                vd = plsc.load_gather(adst_v, [d16])
                vb = plsc.load_gather(bound_v, [d16])
                al = vs + vd
                if has_et:
                    al = al + etb[sl]
                al = jnp.maximum(al, 0.2 * al)
                exrow[sl] = jnp.exp(al - vb)
                return 0
            lax.fori_loop(0, GPS, grp, 0)
            gat.wait()

            def scl(g, _):
                ex = exrow[pl.ds(g * L, L)]
                base = g * L
                for l in range(L):
                    e = ex[l]
                    r = base + l
                    msub[r, pl.ds(0, L)] = msub[r, pl.ds(0, L)] * e
                    msub[r, pl.ds(L, L)] = msub[r, pl.ds(L, L)] * e
                return 0
            lax.fori_loop(0, GPS, scl, 0)

            pltpu.sync_copy(msub, acc_sh.at[dstb], add=True)
            pltpu.sync_copy(exrow, den_sh.at[dstb], add=True)

        plsc.subcore_barrier()
        pltpu.sync_copy(acc_sh.at[pl.ds(s * 128, 128)],
                        acc_out.at[c, pl.ds(s * 128, 128)])
        pltpu.sync_copy(den_sh.at[pl.ds(s * 128, 128)],
                        den_out.at[c, pl.ds(s * 128, 128)])

    scratch = [
        pltpu.VMEM((N,), f32),
        pltpu.VMEM((N,), f32),
        pltpu.VMEM((N,), f32),
        pltpu.VMEM((SUB,), jnp.int32),
    ]
    if has_et:
        scratch.append(pltpu.VMEM((SUB,), f32))
    scratch += [
        pltpu.VMEM((SUB,), jnp.int32),
        pltpu.VMEM((SUB,), jnp.int32),
        pltpu.VMEM((SUB,), jnp.int32),
        pltpu.VMEM((SUB,), jnp.int32),
        pltpu.VMEM((SUB,), f32),
        pltpu.VMEM((SUB, DH), f32),
        pltpu.VMEM_SHARED((N, DH), f32),
        pltpu.VMEM_SHARED((N, DH), f32),
        pltpu.VMEM_SHARED((N,), f32),
        pltpu.SemaphoreType.DMA,
    ]
    return pl.kernel(
        body,
        out_type=[
            jax.ShapeDtypeStruct((NC, N, DH), f32),
            jax.ShapeDtypeStruct((NC, N), f32),
        ],
        mesh=_MESH,
        scratch_types=scratch,
        compiler_params=pltpu.CompilerParams(needs_layout_passes=False, use_tc_tiling_on_sc=False),
    )


_sc_conv1 = _make_sc_conv(True)
_sc_conv2 = _make_sc_conv(False)


def _sc3_body(ap_h, bt_h, srcr_h, dstr_h, r_h, srcb, dstb, bufA, bufB, sem):
    c = lax.axis_index("c")
    s = lax.axis_index("s")
    w = c * NS + s
    for j in range(NSUB):
        row = w * NSUB + j
        pltpu.sync_copy(srcr_h.at[row], srcb)
        pltpu.sync_copy(dstr_h.at[row], dstb)
        ca = pltpu.async_copy(ap_h.at[srcb], bufA, sem)
        ca.wait()
        cb = pltpu.async_copy(bt_h.at[dstb], bufB, sem)
        cb.wait()

        def rb(i, _):
            a0 = bufA[i, pl.ds(0, L)]
            b0 = bufB[i, pl.ds(0, L)]
            bufA[i, pl.ds(0, L)] = jnp.maximum(a0 + b0, 0.0)
            a1 = bufA[i, pl.ds(L, L)]
            b1 = bufB[i, pl.ds(L, L)]
            bufA[i, pl.ds(L, L)] = jnp.maximum(a1 + b1, 0.0)
            return 0
        lax.fori_loop(0, SUB, rb, 0)
        pltpu.sync_copy(bufA, r_h.at[pl.ds(w * EPW + j * SUB, SUB)])


_sc3 = pl.kernel(
    _sc3_body,
    out_type=jax.ShapeDtypeStruct((E, DH), jnp.float32),
    mesh=_MESH,
    scratch_types=[
        pltpu.VMEM((SUB,), jnp.int32),
        pltpu.VMEM((SUB,), jnp.int32),
        pltpu.VMEM((SUB, DH), jnp.float32),
        pltpu.VMEM((SUB, DH), jnp.float32),
        pltpu.SemaphoreType.DMA,
    ],
    compiler_params=pltpu.CompilerParams(needs_layout_passes=False, use_tc_tiling_on_sc=False),
)


def kernel(x, edges, edge_feature, W1, att_src1, att_dst1, We1, att_e1, b1,
           W2, att_src2, att_dst2, b2, Wfc1, bfc1, Wfc2, bfc2):
    f32 = jnp.float32
    h1lin, asrc1, adst1, bound1, exself1, et2d = _tc1(
        x, W1, att_src1.reshape(DH, 1), att_dst1.reshape(DH, 1), We1.T,
        att_e1.reshape(1, DH), edge_feature.T)

    src_rows = edges[0].reshape(ROWS, SUB)
    dst_rows = edges[1].reshape(ROWS, SUB)
    et_rows = et2d.reshape(ROWS, SUB)

    acc1, den1 = _sc_conv1(asrc1.reshape(N), adst1.reshape(N),
                           bound1.reshape(N), h1lin, src_rows, dst_rows,
                           et_rows)
    h2lin, asrc2, adst2, bound2, exself2 = _tc2(
        acc1.reshape(NC * N, DH), den1.reshape(NC * N, 1), exself1, h1lin,
        b1.reshape(1, DH), W2, att_src2.reshape(DH, 1),
        att_dst2.reshape(DH, 1))

    acc2, den2 = _sc_conv2(asrc2.reshape(N), adst2.reshape(N),
                           bound2.reshape(N), h2lin, src_rows, dst_rows)
    ap, bt = _tc3(acc2.reshape(NC * N, DH), den2.reshape(NC * N, 1), exself2,
                  h2lin, b2.reshape(1, DH), Wfc1, bfc1.reshape(1, DH))

    Rm = _sc3(ap, bt, src_rows, dst_rows)

    wd = Wfc2[:, 0] - Wfc2[:, 1]
    k = jnp.arange(128 * DH)
    WD = jnp.where((k[:, None] // DH) == jnp.arange(128)[None, :],
                   jnp.tile(wd, 128)[:, None], 0.0).astype(f32)

    OUT = _tc4(Rm.reshape(512, 128 * DH), WD, bfc1.reshape(1, DH),
               Wfc2, bfc2.reshape(1, 2))
    return OUT.reshape(N * N // 128, 2, 128).transpose(0, 2, 1).reshape(
        N * N, 2)


# speedup vs baseline: 1.1068x; 1.1068x over previous
import jax
import jax.numpy as jnp
from jax import lax
from jax.experimental import pallas as pl
from jax.experimental.pallas import tpu as pltpu
from jax.experimental.pallas import tpu_sc as plsc

N = 2048
E = 65536
DH = 32
NC, NS, L = 2, 16, 16
NW = NC * NS
EPW = E // NW
SUB = 512
NSUB = EPW // SUB
GPS = SUB // L
ROWS = E // SUB

_MESH = plsc.VectorSubcoreMesh(
    core_axis_name="c", subcore_axis_name="s", num_cores=NC, num_subcores=NS)


def _tc1_body(x_ref, w1_ref, asw_ref, adw_ref, wet_ref, ater_ref, eft_ref,
              h1_ref, asrc_ref, adst_ref, bound_ref, exself_ref, et_ref):
    h = jnp.dot(x_ref[...], w1_ref[...], preferred_element_type=jnp.float32)
    h1_ref[...] = h
    asrc = jnp.dot(h, asw_ref[...], preferred_element_type=jnp.float32)
    adst = jnp.dot(h, adw_ref[...], preferred_element_type=jnp.float32)
    wevT = jnp.dot(ater_ref[...], wet_ref[...],
                   preferred_element_type=jnp.float32)
    et = jnp.dot(wevT, eft_ref[...], preferred_element_type=jnp.float32)
    em = jnp.mean(eft_ref[...], axis=1, keepdims=True)
    ets = jnp.dot(wevT, em, preferred_element_type=jnp.float32)[0, 0]
    maxet = jnp.maximum(jnp.max(et), ets)
    b = jnp.max(asrc) + adst + maxet
    bound = jnp.maximum(b, 0.2 * b)
    u = asrc + adst + ets
    exself = jnp.exp(jnp.maximum(u, 0.2 * u) - bound)
    asrc_ref[...] = asrc
    adst_ref[...] = adst
    bound_ref[...] = bound
    exself_ref[...] = exself
    et_ref[...] = et


def _tc1(x, W1, asw, adw, We1_t, ate_row, ef_t):
    f32 = jnp.float32
    return pl.pallas_call(
        _tc1_body,
        out_shape=[
            jax.ShapeDtypeStruct((N, DH), f32),
            jax.ShapeDtypeStruct((N, 1), f32),
            jax.ShapeDtypeStruct((N, 1), f32),
            jax.ShapeDtypeStruct((N, 1), f32),
            jax.ShapeDtypeStruct((N, 1), f32),
            jax.ShapeDtypeStruct((1, E), f32),
        ],
    )(x, W1, asw, adw, We1_t, ate_row, ef_t)


def _tc2_body(acc_ref, den_ref, exs_ref, h1_ref, b1_ref, w2_ref, asw_ref,
              adw_ref, h2_ref, as2_ref, ad2_ref, bo2_ref, ex2_ref):
    a = acc_ref[...]
    accsum = a[0:N] + a[N:2 * N]
    d = den_ref[...]
    densum = d[0:N] + d[N:2 * N]
    exs = exs_ref[...]
    h1l = h1_ref[...]
    h1 = (accsum + exs * h1l) / (densum + exs + 1e-16) + b1_ref[...]
    h2 = jnp.dot(h1, w2_ref[...], preferred_element_type=jnp.float32)
    h2_ref[...] = h2
    as2 = jnp.dot(h2, asw_ref[...], preferred_element_type=jnp.float32)
    ad2 = jnp.dot(h2, adw_ref[...], preferred_element_type=jnp.float32)
    b = jnp.max(as2) + ad2
    bo = jnp.maximum(b, 0.2 * b)
    u = as2 + ad2
    ex2 = jnp.exp(jnp.maximum(u, 0.2 * u) - bo)
    as2_ref[...] = as2
    ad2_ref[...] = ad2
    bo2_ref[...] = bo
    ex2_ref[...] = ex2


def _tc2(acc, den, exs, h1lin, b1r, W2, asw2, adw2):
    f32 = jnp.float32
    return pl.pallas_call(
        _tc2_body,
        out_shape=[
            jax.ShapeDtypeStruct((N, DH), f32),
            jax.ShapeDtypeStruct((N, 1), f32),
            jax.ShapeDtypeStruct((N, 1), f32),
            jax.ShapeDtypeStruct((N, 1), f32),
            jax.ShapeDtypeStruct((N, 1), f32),
        ],
    )(acc, den, exs, h1lin, b1r, W2, asw2, adw2)


def _tc3_body(acc_ref, den_ref, exs_ref, h2_ref, b2_ref, wfc1_ref, bfc1_ref,
              ap_ref, bt_ref):
    a = acc_ref[...]
    accsum = a[0:N] + a[N:2 * N]
    d = den_ref[...]
    densum = d[0:N] + d[N:2 * N]
    exs = exs_ref[...]
    h2l = h2_ref[...]
    h2 = (accsum + exs * h2l) / (densum + exs + 1e-16) + b2_ref[...]
    w = wfc1_ref[...]
    ap_ref[...] = jnp.dot(h2, w[0:DH], preferred_element_type=jnp.float32) \
        + bfc1_ref[...]
    bt_ref[...] = jnp.dot(h2, w[DH:2 * DH], preferred_element_type=jnp.float32)


def _tc3(acc, den, exs, h2lin, b2r, Wfc1, bfc1r):
    f32 = jnp.float32
    return pl.pallas_call(
        _tc3_body,
        out_shape=[
            jax.ShapeDtypeStruct((N, DH), f32),
            jax.ShapeDtypeStruct((N, DH), f32),
        ],
    )(acc, den, exs, h2lin, b2r, Wfc1, bfc1r)


_TC4_BLK = 2048


def _tc4_body(t_ref, bfc1_ref, wfc2_ref, bfc2_ref, out_ref):
    pid = pl.program_id(0)
    bfc2v = bfc2_ref[...]
    z = jnp.dot(jnp.maximum(bfc1_ref[...], 0.0), wfc2_ref[...],
                preferred_element_type=jnp.float32) + bfc2v
    ez = jnp.exp(z - jnp.max(z))
    yc = ez / jnp.sum(ez)
    c0 = yc[0, 0]
    c1 = yc[0, 1]

    @pl.when(pid == 0)
    def _edge():
        db = bfc2v[0, 0] - bfc2v[0, 1]
        tt = t_ref[...] + db
        y0 = 1.0 / (1.0 + jnp.exp(-tt))
        y1 = 1.0 / (1.0 + jnp.exp(tt))
        epart = jnp.concatenate([y0[:, None, :], y1[:, None, :]], axis=1)
        nc = _TC4_BLK - 512
        cpart = jnp.concatenate(
            [jnp.full((nc, 1, 128), c0, jnp.float32),
             jnp.full((nc, 1, 128), c1, jnp.float32)], axis=1)
        out_ref[...] = jnp.concatenate([epart, cpart], axis=0)

    @pl.when(pid != 0)
    def _fill():
        out_ref[...] = jnp.concatenate(
            [jnp.full((_TC4_BLK, 1, 128), c0, jnp.float32),
             jnp.full((_TC4_BLK, 1, 128), c1, jnp.float32)], axis=1)


def _tc4(t2d, bfc1r, Wfc2, bfc2r):
    cmap = lambda i: (0, 0)
    return pl.pallas_call(
        _tc4_body,
        grid=(N * N // 128 // _TC4_BLK,),
        in_specs=[
            pl.BlockSpec((512, 128), cmap),
            pl.BlockSpec((1, DH), cmap),
            pl.BlockSpec((DH, 2), cmap),
            pl.BlockSpec((1, 2), cmap),
        ],
        out_specs=pl.BlockSpec((_TC4_BLK, 2, 128), lambda i: (i, 0, 0)),
        out_shape=jax.ShapeDtypeStruct((N * N // 128, 2, 128), jnp.float32),
    )(t2d, bfc1r, Wfc2, bfc2r)


def _make_sc_conv(has_et):
    f32 = jnp.float32

    def body(*refs):
        if has_et:
            (asrc_h, adst_h, bound_h, h_h, srcr_h, dstr_h, etr_h,
             acc_out, den_out,
             asrc_v, adst_v, bound_v, srcb, etb, db0, db1, db2, db3,
             exrow, msub, h_sh, acc_sh, den_sh, sem) = refs
        else:
            (asrc_h, adst_h, bound_h, h_h, srcr_h, dstr_h,
             acc_out, den_out,
             asrc_v, adst_v, bound_v, srcb, db0, db1, db2, db3,
             exrow, msub, h_sh, acc_sh, den_sh, sem) = refs
            etb = None
        dstbs = (db0, db1, db2, db3)
        c = lax.axis_index("c")
        s = lax.axis_index("s")
        w = c * NS + s
        z16 = jnp.zeros((L,), f32)

        def zb(i, _):
            msub[i, pl.ds(0, L)] = z16
            msub[i, pl.ds(L, L)] = z16
            return 0
        lax.fori_loop(0, 128, zb, 0)
        for k in range(128 // L):
            exrow[pl.ds(k * L, L)] = z16
        pltpu.sync_copy(msub.at[pl.ds(0, 128)], acc_sh.at[pl.ds(s * 128, 128)])
        pltpu.sync_copy(exrow.at[pl.ds(0, 128)], den_sh.at[pl.ds(s * 128, 128)])
        pltpu.sync_copy(h_h.at[pl.ds(s * 128, 128)],
                        h_sh.at[pl.ds(s * 128, 128)])
        pltpu.sync_copy(asrc_h, asrc_v)
        pltpu.sync_copy(adst_h, adst_v)
        pltpu.sync_copy(bound_h, bound_v)
        plsc.subcore_barrier()

        for j in range(NSUB):
            row = w * NSUB + j
            dstb = dstbs[j]
            pltpu.sync_copy(srcr_h.at[row], srcb)
            pltpu.sync_copy(dstr_h.at[row], dstb)
            if has_et:
                pltpu.sync_copy(etr_h.at[row], etb)
            gat = pltpu.async_copy(h_sh.at[srcb], msub, sem)

            def grp(g, _):
                sl = pl.ds(g * L, L)
                s16 = srcb[sl]
                d16 = dstb[sl]
                vs = plsc.load_gather(asrc_v, [s16])
                vd = plsc.load_gather(adst_v, [d16])
                vb = plsc.load_gather(bound_v, [d16])
                al = vs + vd
                if has_et:
                    al = al + etb[sl]
                al = jnp.maximum(al, 0.2 * al)
                exrow[sl] = jnp.exp(al - vb)
                return 0
            lax.fori_loop(0, GPS, grp, 0)
            gat.wait()

            def scl(g, _):
                ex = exrow[pl.ds(g * L, L)]
                base = g * L
                for l in range(L):
                    e = ex[l]
                    r = base + l
                    msub[r, pl.ds(0, L)] = msub[r, pl.ds(0, L)] * e
                    msub[r, pl.ds(L, L)] = msub[r, pl.ds(L, L)] * e
                return 0
            lax.fori_loop(0, GPS, scl, 0)

            pltpu.sync_copy(msub, acc_sh.at[dstb], add=True)
            pltpu.sync_copy(exrow, den_sh.at[dstb], add=True)

        plsc.subcore_barrier()
        pltpu.sync_copy(acc_sh.at[pl.ds(s * 128, 128)],
                        acc_out.at[c, pl.ds(s * 128, 128)])
        pltpu.sync_copy(den_sh.at[pl.ds(s * 128, 128)],
                        den_out.at[c, pl.ds(s * 128, 128)])

    scratch = [
        pltpu.VMEM((N,), f32),
        pltpu.VMEM((N,), f32),
        pltpu.VMEM((N,), f32),
        pltpu.VMEM((SUB,), jnp.int32),
    ]
    if has_et:
        scratch.append(pltpu.VMEM((SUB,), f32))
    scratch += [
        pltpu.VMEM((SUB,), jnp.int32),
        pltpu.VMEM((SUB,), jnp.int32),
        pltpu.VMEM((SUB,), jnp.int32),
        pltpu.VMEM((SUB,), jnp.int32),
        pltpu.VMEM((SUB,), f32),
        pltpu.VMEM((SUB, DH), f32),
        pltpu.VMEM_SHARED((N, DH), f32),
        pltpu.VMEM_SHARED((N, DH), f32),
        pltpu.VMEM_SHARED((N,), f32),
        pltpu.SemaphoreType.DMA,
    ]
    return pl.kernel(
        body,
        out_type=[
            jax.ShapeDtypeStruct((NC, N, DH), f32),
            jax.ShapeDtypeStruct((NC, N), f32),
        ],
        mesh=_MESH,
        scratch_types=scratch,
        compiler_params=pltpu.CompilerParams(needs_layout_passes=False, use_tc_tiling_on_sc=False),
    )


_sc_conv1 = _make_sc_conv(True)
_sc_conv2 = _make_sc_conv(False)


def _sc3_body(ap_h, bt_h, srcr_h, dstr_h, wd_h, t_h,
              srcb, dstb, bufA, bufB, wdb, tbuf, sem):
    c = lax.axis_index("c")
    s = lax.axis_index("s")
    w = c * NS + s
    pltpu.sync_copy(wd_h, wdb)
    wv0 = wdb[pl.ds(0, L)]
    wv1 = wdb[pl.ds(L, L)]
    for j in range(NSUB):
        row = w * NSUB + j
        pltpu.sync_copy(srcr_h.at[row], srcb)
        pltpu.sync_copy(dstr_h.at[row], dstb)
        ca = pltpu.async_copy(ap_h.at[srcb], bufA, sem)
        ca.wait()
        cb = pltpu.async_copy(bt_h.at[dstb], bufB, sem)
        cb.wait()

        def grp(g, _):
            r16 = g * L + lax.iota(jnp.int32, L)
            acc = jnp.zeros((L,), jnp.float32)
            for dd in range(DH):
                cd = jnp.full((L,), dd, jnp.int32)
                va = plsc.load_gather(bufA, [r16, cd])
                vb = plsc.load_gather(bufB, [r16, cd])
                e = wv0[dd] if dd < L else wv1[dd - L]
                acc = acc + jnp.maximum(va + vb, 0.0) * e
            tbuf[pl.ds(g * L, L)] = acc
            return 0
        lax.fori_loop(0, GPS, grp, 0)
        pltpu.sync_copy(tbuf, t_h.at[pl.ds(w * EPW + j * SUB, SUB)])


_sc3 = pl.kernel(
    _sc3_body,
    out_type=jax.ShapeDtypeStruct((E,), jnp.float32),
    mesh=_MESH,
    scratch_types=[
        pltpu.VMEM((SUB,), jnp.int32),
        pltpu.VMEM((SUB,), jnp.int32),
        pltpu.VMEM((SUB, DH), jnp.float32),
        pltpu.VMEM((SUB, DH), jnp.float32),
        pltpu.VMEM((DH,), jnp.float32),
        pltpu.VMEM((SUB,), jnp.float32),
        pltpu.SemaphoreType.DMA,
    ],
    compiler_params=pltpu.CompilerParams(needs_layout_passes=False, use_tc_tiling_on_sc=False),
)


def kernel(x, edges, edge_feature, W1, att_src1, att_dst1, We1, att_e1, b1,
           W2, att_src2, att_dst2, b2, Wfc1, bfc1, Wfc2, bfc2):
    f32 = jnp.float32
    h1lin, asrc1, adst1, bound1, exself1, et2d = _tc1(
        x, W1, att_src1.reshape(DH, 1), att_dst1.reshape(DH, 1), We1.T,
        att_e1.reshape(1, DH), edge_feature.T)

    src_rows = edges[0].reshape(ROWS, SUB)
    dst_rows = edges[1].reshape(ROWS, SUB)
    et_rows = et2d.reshape(ROWS, SUB)

    acc1, den1 = _sc_conv1(asrc1.reshape(N), adst1.reshape(N),
                           bound1.reshape(N), h1lin, src_rows, dst_rows,
                           et_rows)
    h2lin, asrc2, adst2, bound2, exself2 = _tc2(
        acc1.reshape(NC * N, DH), den1.reshape(NC * N, 1), exself1, h1lin,
        b1.reshape(1, DH), W2, att_src2.reshape(DH, 1),
        att_dst2.reshape(DH, 1))

    acc2, den2 = _sc_conv2(asrc2.reshape(N), adst2.reshape(N),
                           bound2.reshape(N), h2lin, src_rows, dst_rows)
    ap, bt = _tc3(acc2.reshape(NC * N, DH), den2.reshape(NC * N, 1), exself2,
                  h2lin, b2.reshape(1, DH), Wfc1, bfc1.reshape(1, DH))

    wd = (Wfc2[:, 0] - Wfc2[:, 1]).astype(f32)
    tvec = _sc3(ap, bt, src_rows, dst_rows, wd)

    OUT = _tc4(tvec.reshape(512, 128), bfc1.reshape(1, DH), Wfc2,
               bfc2.reshape(1, 2))
    return OUT.transpose(0, 2, 1).reshape(N * N, 2)


# speedup vs baseline: 1.3649x; 1.2332x over previous
import jax
import jax.numpy as jnp
from jax import lax
from jax.experimental import pallas as pl
from jax.experimental.pallas import tpu as pltpu
from jax.experimental.pallas import tpu_sc as plsc

N = 2048
E = 65536
DH = 32
NC, NS, L = 2, 16, 16
NW = NC * NS
EPW = E // NW
SUB = 512
NSUB = EPW // SUB
GPS = SUB // L
ROWS = E // SUB

_MESH = plsc.VectorSubcoreMesh(
    core_axis_name="c", subcore_axis_name="s", num_cores=NC, num_subcores=NS)


def _tc1_body(x_ref, w1_ref, asw_ref, adw_ref, wet_ref, ater_ref, eft_ref,
              h1_ref, asrc_ref, adst_ref, bound_ref, exself_ref, et_ref):
    h = jnp.dot(x_ref[...], w1_ref[...], preferred_element_type=jnp.float32)
    h1_ref[...] = h
    asrc = jnp.dot(h, asw_ref[...], preferred_element_type=jnp.float32)
    adst = jnp.dot(h, adw_ref[...], preferred_element_type=jnp.float32)
    wevT = jnp.dot(ater_ref[...], wet_ref[...],
                   preferred_element_type=jnp.float32)
    et = jnp.dot(wevT, eft_ref[...], preferred_element_type=jnp.float32)
    em = jnp.mean(eft_ref[...], axis=1, keepdims=True)
    ets = jnp.dot(wevT, em, preferred_element_type=jnp.float32)[0, 0]
    maxet = jnp.maximum(jnp.max(et), ets)
    b = jnp.max(asrc) + adst + maxet
    bound = jnp.maximum(b, 0.2 * b)
    u = asrc + adst + ets
    exself = jnp.exp(jnp.maximum(u, 0.2 * u) - bound)
    asrc_ref[...] = asrc
    adst_ref[...] = adst
    bound_ref[...] = bound
    exself_ref[...] = exself
    et_ref[...] = et


def _tc1(x, W1, asw, adw, We1_t, ate_row, ef_t):
    f32 = jnp.float32
    return pl.pallas_call(
        _tc1_body,
        out_shape=[
            jax.ShapeDtypeStruct((N, DH), f32),
            jax.ShapeDtypeStruct((N, 1), f32),
            jax.ShapeDtypeStruct((N, 1), f32),
            jax.ShapeDtypeStruct((N, 1), f32),
            jax.ShapeDtypeStruct((N, 1), f32),
            jax.ShapeDtypeStruct((1, E), f32),
        ],
    )(x, W1, asw, adw, We1_t, ate_row, ef_t)


def _tc2_body(acc_ref, den_ref, exs_ref, h1_ref, b1_ref, w2_ref, asw_ref,
              adw_ref, h2_ref, as2_ref, ad2_ref, bo2_ref, ex2_ref):
    a = acc_ref[...]
    accsum = a[0:N] + a[N:2 * N]
    d = den_ref[...]
    densum = d[0:N] + d[N:2 * N]
    exs = exs_ref[...]
    h1l = h1_ref[...]
    h1 = (accsum + exs * h1l) / (densum + exs + 1e-16) + b1_ref[...]
    h2 = jnp.dot(h1, w2_ref[...], preferred_element_type=jnp.float32)
    h2_ref[...] = h2
    as2 = jnp.dot(h2, asw_ref[...], preferred_element_type=jnp.float32)
    ad2 = jnp.dot(h2, adw_ref[...], preferred_element_type=jnp.float32)
    b = jnp.max(as2) + ad2
    bo = jnp.maximum(b, 0.2 * b)
    u = as2 + ad2
    ex2 = jnp.exp(jnp.maximum(u, 0.2 * u) - bo)
    as2_ref[...] = as2
    ad2_ref[...] = ad2
    bo2_ref[...] = bo
    ex2_ref[...] = ex2


def _tc2(acc, den, exs, h1lin, b1r, W2, asw2, adw2):
    f32 = jnp.float32
    return pl.pallas_call(
        _tc2_body,
        out_shape=[
            jax.ShapeDtypeStruct((N, DH), f32),
            jax.ShapeDtypeStruct((N, 1), f32),
            jax.ShapeDtypeStruct((N, 1), f32),
            jax.ShapeDtypeStruct((N, 1), f32),
            jax.ShapeDtypeStruct((N, 1), f32),
        ],
    )(acc, den, exs, h1lin, b1r, W2, asw2, adw2)


def _tc3_body(acc_ref, den_ref, exs_ref, h2_ref, b2_ref, wfc1_ref, bfc1_ref,
              ap_ref, bt_ref):
    a = acc_ref[...]
    accsum = a[0:N] + a[N:2 * N]
    d = den_ref[...]
    densum = d[0:N] + d[N:2 * N]
    exs = exs_ref[...]
    h2l = h2_ref[...]
    h2 = (accsum + exs * h2l) / (densum + exs + 1e-16) + b2_ref[...]
    w = wfc1_ref[...]
    ap_ref[...] = jnp.dot(h2, w[0:DH], preferred_element_type=jnp.float32) \
        + bfc1_ref[...]
    bt_ref[...] = jnp.dot(h2, w[DH:2 * DH], preferred_element_type=jnp.float32)


def _tc3(acc, den, exs, h2lin, b2r, Wfc1, bfc1r):
    f32 = jnp.float32
    return pl.pallas_call(
        _tc3_body,
        out_shape=[
            jax.ShapeDtypeStruct((N, DH), f32),
            jax.ShapeDtypeStruct((N, DH), f32),
        ],
    )(acc, den, exs, h2lin, b2r, Wfc1, bfc1r)


_TC4_BLK = 2048


def _tc4_body(t_ref, bfc1_ref, wfc2_ref, bfc2_ref, out_ref):
    pid = pl.program_id(0)
    bfc2v = bfc2_ref[...]
    z = jnp.dot(jnp.maximum(bfc1_ref[...], 0.0), wfc2_ref[...],
                preferred_element_type=jnp.float32) + bfc2v
    ez = jnp.exp(z - jnp.max(z))
    yc = ez / jnp.sum(ez)
    c0 = yc[0, 0]
    c1 = yc[0, 1]

    @pl.when(pid == 0)
    def _edge():
        db = bfc2v[0, 0] - bfc2v[0, 1]
        tt = t_ref[...] + db
        y0 = 1.0 / (1.0 + jnp.exp(-tt))
        y1 = 1.0 / (1.0 + jnp.exp(tt))
        epart = jnp.concatenate([y0[:, None, :], y1[:, None, :]], axis=1)
        nc = _TC4_BLK - 512
        cpart = jnp.concatenate(
            [jnp.full((nc, 1, 128), c0, jnp.float32),
             jnp.full((nc, 1, 128), c1, jnp.float32)], axis=1)
        out_ref[...] = jnp.concatenate([epart, cpart], axis=0)

    @pl.when(pid != 0)
    def _fill():
        out_ref[...] = jnp.concatenate(
            [jnp.full((_TC4_BLK, 1, 128), c0, jnp.float32),
             jnp.full((_TC4_BLK, 1, 128), c1, jnp.float32)], axis=1)


def _tc4(t2d, bfc1r, Wfc2, bfc2r):
    cmap = lambda i: (0, 0)
    return pl.pallas_call(
        _tc4_body,
        grid=(N * N // 128 // _TC4_BLK,),
        in_specs=[
            pl.BlockSpec((512, 128), cmap),
            pl.BlockSpec((1, DH), cmap),
            pl.BlockSpec((DH, 2), cmap),
            pl.BlockSpec((1, 2), cmap),
        ],
        out_specs=pl.BlockSpec((_TC4_BLK, 2, 128), lambda i: (i, 0, 0)),
        out_shape=jax.ShapeDtypeStruct((N * N // 128, 2, 128), jnp.float32),
    )(t2d, bfc1r, Wfc2, bfc2r)


def _make_sc_conv(has_et):
    f32 = jnp.float32

    def body(*refs):
        if has_et:
            (asrc_h, adst_h, bound_h, h_h, srcr_h, dstr_h, etr_h,
             acc_out, den_out,
             asrc_v, adst_v, bound_v, srcb, etb, db0, db1, db2, db3,
             exrow, msub, h_sh, acc_sh, den_sh, sem) = refs
        else:
            (asrc_h, adst_h, bound_h, h_h, srcr_h, dstr_h,
             acc_out, den_out,
             asrc_v, adst_v, bound_v, srcb, db0, db1, db2, db3,
             exrow, msub, h_sh, acc_sh, den_sh, sem) = refs
            etb = None
        dstbs = (db0, db1, db2, db3)
        c = lax.axis_index("c")
        s = lax.axis_index("s")
        w = c * NS + s
        z16 = jnp.zeros((L,), f32)

        def zb(i, _):
            msub[i, pl.ds(0, L)] = z16
            msub[i, pl.ds(L, L)] = z16
            return 0
        lax.fori_loop(0, 128, zb, 0)
        for k in range(128 // L):
            exrow[pl.ds(k * L, L)] = z16
        pltpu.sync_copy(msub.at[pl.ds(0, 128)], acc_sh.at[pl.ds(s * 128, 128)])
        pltpu.sync_copy(exrow.at[pl.ds(0, 128)], den_sh.at[pl.ds(s * 128, 128)])
        pltpu.sync_copy(h_h.at[pl.ds(s * 128, 128)],
                        h_sh.at[pl.ds(s * 128, 128)])
        pltpu.sync_copy(asrc_h, asrc_v)
        pltpu.sync_copy(adst_h, adst_v)
        pltpu.sync_copy(bound_h, bound_v)
        plsc.subcore_barrier()

        for j in range(NSUB):
            row = w * NSUB + j
            dstb = dstbs[j]
            pltpu.sync_copy(srcr_h.at[row], srcb)
            pltpu.sync_copy(dstr_h.at[row], dstb)
            if has_et:
                pltpu.sync_copy(etr_h.at[row], etb)
            gat = pltpu.async_copy(h_sh.at[srcb], msub, sem)

            def grp(g, _):
                sl = pl.ds(g * L, L)
                s16 = srcb[sl]
                d16 = dstb[sl]
                vs = plsc.load_gather(asrc_v, [s16])
                vd = plsc.load_gather(adst_v, [d16])
                vb = plsc.load_gather(bound_v, [d16])
                al = vs + vd
                if has_et:
                    al = al + etb[sl]
                al = jnp.maximum(al, 0.2 * al)
                exrow[sl] = jnp.exp(al - vb)
                return 0
            lax.fori_loop(0, GPS, grp, 0)
            gat.wait()

            def scl(g, _):
                ex = exrow[pl.ds(g * L, L)]
                base = g * L
                for l in range(L):
                    e = ex[l]
                    r = base + l
                    msub[r, pl.ds(0, L)] = msub[r, pl.ds(0, L)] * e
                    msub[r, pl.ds(L, L)] = msub[r, pl.ds(L, L)] * e
                return 0
            lax.fori_loop(0, GPS, scl, 0)

            pltpu.sync_copy(msub, acc_sh.at[dstb], add=True)
            pltpu.sync_copy(exrow, den_sh.at[dstb], add=True)

        plsc.subcore_barrier()
        pltpu.sync_copy(acc_sh.at[pl.ds(s * 128, 128)],
                        acc_out.at[c, pl.ds(s * 128, 128)])
        pltpu.sync_copy(den_sh.at[pl.ds(s * 128, 128)],
                        den_out.at[c, pl.ds(s * 128, 128)])

    scratch = [
        pltpu.VMEM((N,), f32),
        pltpu.VMEM((N,), f32),
        pltpu.VMEM((N,), f32),
        pltpu.VMEM((SUB,), jnp.int32),
    ]
    if has_et:
        scratch.append(pltpu.VMEM((SUB,), f32))
    scratch += [
        pltpu.VMEM((SUB,), jnp.int32),
        pltpu.VMEM((SUB,), jnp.int32),
        pltpu.VMEM((SUB,), jnp.int32),
        pltpu.VMEM((SUB,), jnp.int32),
        pltpu.VMEM((SUB,), f32),
        pltpu.VMEM((SUB, DH), f32),
        pltpu.VMEM_SHARED((N, DH), f32),
        pltpu.VMEM_SHARED((N, DH), f32),
        pltpu.VMEM_SHARED((N,), f32),
        pltpu.SemaphoreType.DMA,
    ]
    return pl.kernel(
        body,
        out_type=[
            jax.ShapeDtypeStruct((NC, N, DH), f32),
            jax.ShapeDtypeStruct((NC, N), f32),
        ],
        mesh=_MESH,
        scratch_types=scratch,
        compiler_params=pltpu.CompilerParams(needs_layout_passes=False, use_tc_tiling_on_sc=False),
    )


_sc_conv1 = _make_sc_conv(True)
_sc_conv2 = _make_sc_conv(False)


def _sc3_body(ap_h, bt_h, srcr_h, dstr_h, wd_h, t_h,
              srcb, dstb, bufA, bufB, wdb, tbuf, tbx, sem):
    c = lax.axis_index("c")
    s = lax.axis_index("s")
    w = c * NS + s
    pltpu.sync_copy(wd_h, wdb)
    wv0 = wdb[pl.ds(0, L)]
    wv1 = wdb[pl.ds(L, L)]
    for j in range(NSUB):
        row = w * NSUB + j
        pltpu.sync_copy(srcr_h.at[row], srcb)
        pltpu.sync_copy(dstr_h.at[row], dstb)
        ca = pltpu.async_copy(ap_h.at[srcb], bufA, sem)
        ca.wait()
        cb = pltpu.async_copy(bt_h.at[dstb], bufB, sem)
        cb.wait()

        def grp(g, _):
            base = g * L
            for l in range(L):
                r = base + l
                a0 = bufA[r, pl.ds(0, L)]
                b0 = bufB[r, pl.ds(0, L)]
                a1 = bufA[r, pl.ds(L, L)]
                b1 = bufB[r, pl.ds(L, L)]
                sv = (jnp.maximum(a0 + b0, 0.0) * wv0
                      + jnp.maximum(a1 + b1, 0.0) * wv1)
                tbx[pl.ds(l * (L + 1), L)] = sv
            acc = jnp.zeros((L,), jnp.float32)
            rows17 = lax.iota(jnp.int32, L) * (L + 1)
            for cc in range(L):
                acc = acc + plsc.load_gather(tbx, [rows17 + cc])
            tbuf[pl.ds(base, L)] = acc
            return 0
        lax.fori_loop(0, GPS, grp, 0)
        pltpu.sync_copy(tbuf, t_h.at[pl.ds(w * EPW + j * SUB, SUB)])


_sc3 = pl.kernel(
    _sc3_body,
    out_type=jax.ShapeDtypeStruct((E,), jnp.float32),
    mesh=_MESH,
    scratch_types=[
        pltpu.VMEM((SUB,), jnp.int32),
        pltpu.VMEM((SUB,), jnp.int32),
        pltpu.VMEM((SUB, DH), jnp.float32),
        pltpu.VMEM((SUB, DH), jnp.float32),
        pltpu.VMEM((DH,), jnp.float32),
        pltpu.VMEM((SUB,), jnp.float32),
        pltpu.VMEM((L * (L + 1),), jnp.float32),
        pltpu.SemaphoreType.DMA,
    ],
    compiler_params=pltpu.CompilerParams(needs_layout_passes=False, use_tc_tiling_on_sc=False),
)


def kernel(x, edges, edge_feature, W1, att_src1, att_dst1, We1, att_e1, b1,
           W2, att_src2, att_dst2, b2, Wfc1, bfc1, Wfc2, bfc2):
    f32 = jnp.float32
    h1lin, asrc1, adst1, bound1, exself1, et2d = _tc1(
        x, W1, att_src1.reshape(DH, 1), att_dst1.reshape(DH, 1), We1.T,
        att_e1.reshape(1, DH), edge_feature.T)

    src_rows = edges[0].reshape(ROWS, SUB)
    dst_rows = edges[1].reshape(ROWS, SUB)
    et_rows = et2d.reshape(ROWS, SUB)

    acc1, den1 = _sc_conv1(asrc1.reshape(N), adst1.reshape(N),
                           bound1.reshape(N), h1lin, src_rows, dst_rows,
                           et_rows)
    h2lin, asrc2, adst2, bound2, exself2 = _tc2(
        acc1.reshape(NC * N, DH), den1.reshape(NC * N, 1), exself1, h1lin,
        b1.reshape(1, DH), W2, att_src2.reshape(DH, 1),
        att_dst2.reshape(DH, 1))

    acc2, den2 = _sc_conv2(asrc2.reshape(N), adst2.reshape(N),
                           bound2.reshape(N), h2lin, src_rows, dst_rows)
    ap, bt = _tc3(acc2.reshape(NC * N, DH), den2.reshape(NC * N, 1), exself2,
                  h2lin, b2.reshape(1, DH), Wfc1, bfc1.reshape(1, DH))

    wd = (Wfc2[:, 0] - Wfc2[:, 1]).astype(f32)
    tvec = _sc3(ap, bt, src_rows, dst_rows, wd)

    OUT = _tc4(tvec.reshape(512, 128), bfc1.reshape(1, DH), Wfc2,
               bfc2.reshape(1, 2))
    return OUT.transpose(0, 2, 1).reshape(N * N, 2)


# speedup vs baseline: 1.4066x; 1.0306x over previous
import jax
import jax.numpy as jnp
from jax import lax
from jax.experimental import pallas as pl
from jax.experimental.pallas import tpu as pltpu
from jax.experimental.pallas import tpu_sc as plsc

N = 2048
E = 65536
DH = 32
NC, NS, L = 2, 16, 16
NW = NC * NS
EPW = E // NW
SUB = 512
NSUB = EPW // SUB
GPS = SUB // L
ROWS = E // SUB

_MESH = plsc.VectorSubcoreMesh(
    core_axis_name="c", subcore_axis_name="s", num_cores=NC, num_subcores=NS)


def _tc1_body(x_ref, w1_ref, asw_ref, adw_ref, wet_ref, ater_ref, eft_ref,
              h1_ref, asrc_ref, adst_ref, bound_ref, exself_ref, et_ref):
    h = jnp.dot(x_ref[...], w1_ref[...], preferred_element_type=jnp.float32)
    h1_ref[...] = h
    asrc = jnp.dot(h, asw_ref[...], preferred_element_type=jnp.float32)
    adst = jnp.dot(h, adw_ref[...], preferred_element_type=jnp.float32)
    wevT = jnp.dot(ater_ref[...], wet_ref[...],
                   preferred_element_type=jnp.float32)
    et = jnp.dot(wevT, eft_ref[...], preferred_element_type=jnp.float32)
    em = jnp.mean(eft_ref[...], axis=1, keepdims=True)
    ets = jnp.dot(wevT, em, preferred_element_type=jnp.float32)[0, 0]
    maxet = jnp.maximum(jnp.max(et), ets)
    b = jnp.max(asrc) + adst + maxet
    bound = jnp.maximum(b, 0.2 * b)
    u = asrc + adst + ets
    exself = jnp.exp(jnp.maximum(u, 0.2 * u) - bound)
    asrc_ref[...] = asrc[:, 0]
    adst_ref[...] = adst[:, 0]
    bound_ref[...] = bound[:, 0]
    exself_ref[...] = exself
    et_ref[...] = et


def _tc1(x, W1, asw, adw, We1_t, ate_row, ef_t):
    f32 = jnp.float32
    return pl.pallas_call(
        _tc1_body,
        out_shape=[
            jax.ShapeDtypeStruct((N, DH), f32),
            jax.ShapeDtypeStruct((N,), f32),
            jax.ShapeDtypeStruct((N,), f32),
            jax.ShapeDtypeStruct((N,), f32),
            jax.ShapeDtypeStruct((N, 1), f32),
            jax.ShapeDtypeStruct((1, E), f32),
        ],
    )(x, W1, asw, adw, We1_t, ate_row, ef_t)


def _tc2_body(acc_ref, den_ref, exs_ref, h1_ref, b1_ref, w2_ref, asw_ref,
              adw_ref, h2_ref, as2_ref, ad2_ref, bo2_ref, ex2_ref):
    a = acc_ref[...]
    accsum = a[0:N] + a[N:2 * N]
    d = den_ref[...]
    densum = d[0:N] + d[N:2 * N]
    exs = exs_ref[...]
    h1l = h1_ref[...]
    h1 = (accsum + exs * h1l) / (densum + exs + 1e-16) + b1_ref[...]
    h2 = jnp.dot(h1, w2_ref[...], preferred_element_type=jnp.float32)
    h2_ref[...] = h2
    as2 = jnp.dot(h2, asw_ref[...], preferred_element_type=jnp.float32)
    ad2 = jnp.dot(h2, adw_ref[...], preferred_element_type=jnp.float32)
    b = jnp.max(as2) + ad2
    bo = jnp.maximum(b, 0.2 * b)
    u = as2 + ad2
    ex2 = jnp.exp(jnp.maximum(u, 0.2 * u) - bo)
    as2_ref[...] = as2[:, 0]
    ad2_ref[...] = ad2[:, 0]
    bo2_ref[...] = bo[:, 0]
    ex2_ref[...] = ex2


def _tc2(acc, den, exs, h1lin, b1r, W2, asw2, adw2):
    f32 = jnp.float32
    return pl.pallas_call(
        _tc2_body,
        out_shape=[
            jax.ShapeDtypeStruct((N, DH), f32),
            jax.ShapeDtypeStruct((N,), f32),
            jax.ShapeDtypeStruct((N,), f32),
            jax.ShapeDtypeStruct((N,), f32),
            jax.ShapeDtypeStruct((N, 1), f32),
        ],
    )(acc, den, exs, h1lin, b1r, W2, asw2, adw2)


def _tc3_body(acc_ref, den_ref, exs_ref, h2_ref, b2_ref, wfc1_ref, bfc1_ref,
              ap_ref, bt_ref):
    a = acc_ref[...]
    accsum = a[0:N] + a[N:2 * N]
    d = den_ref[...]
    densum = d[0:N] + d[N:2 * N]
    exs = exs_ref[...]
    h2l = h2_ref[...]
    h2 = (accsum + exs * h2l) / (densum + exs + 1e-16) + b2_ref[...]
    w = wfc1_ref[...]
    ap_ref[...] = jnp.dot(h2, w[0:DH], preferred_element_type=jnp.float32) \
        + bfc1_ref[...]
    bt_ref[...] = jnp.dot(h2, w[DH:2 * DH], preferred_element_type=jnp.float32)


def _tc3(acc, den, exs, h2lin, b2r, Wfc1, bfc1r):
    f32 = jnp.float32
    return pl.pallas_call(
        _tc3_body,
        out_shape=[
            jax.ShapeDtypeStruct((N, DH), f32),
            jax.ShapeDtypeStruct((N, DH), f32),
        ],
    )(acc, den, exs, h2lin, b2r, Wfc1, bfc1r)


_TC4_BLK = 2048


def _tc4_body(t_ref, bfc1_ref, wfc2_ref, bfc2_ref, out_ref):
    pid = pl.program_id(0)
    bfc2v = bfc2_ref[...]
    z = jnp.dot(jnp.maximum(bfc1_ref[...], 0.0), wfc2_ref[...],
                preferred_element_type=jnp.float32) + bfc2v
    ez = jnp.exp(z - jnp.max(z))
    yc = ez / jnp.sum(ez)
    c0 = yc[0, 0]
    c1 = yc[0, 1]

    @pl.when(pid == 0)
    def _edge():
        db = bfc2v[0, 0] - bfc2v[0, 1]
        tt = t_ref[...] + db
        y0 = 1.0 / (1.0 + jnp.exp(-tt))
        y1 = 1.0 / (1.0 + jnp.exp(tt))
        epart = jnp.concatenate([y0[:, None, :], y1[:, None, :]], axis=1)
        nc = _TC4_BLK - 512
        cpart = jnp.concatenate(
            [jnp.full((nc, 1, 128), c0, jnp.float32),
             jnp.full((nc, 1, 128), c1, jnp.float32)], axis=1)
        out_ref[...] = jnp.concatenate([epart, cpart], axis=0)

    @pl.when(pid != 0)
    def _fill():
        out_ref[...] = jnp.concatenate(
            [jnp.full((_TC4_BLK, 1, 128), c0, jnp.float32),
             jnp.full((_TC4_BLK, 1, 128), c1, jnp.float32)], axis=1)


def _tc4(t2d, bfc1r, Wfc2, bfc2r):
    cmap = lambda i: (0, 0)
    return pl.pallas_call(
        _tc4_body,
        grid=(N * N // 128 // _TC4_BLK,),
        in_specs=[
            pl.BlockSpec((512, 128), cmap),
            pl.BlockSpec((1, DH), cmap),
            pl.BlockSpec((DH, 2), cmap),
            pl.BlockSpec((1, 2), cmap),
        ],
        out_specs=pl.BlockSpec((_TC4_BLK, 2, 128), lambda i: (i, 0, 0)),
        out_shape=jax.ShapeDtypeStruct((N * N // 128, 2, 128), jnp.float32),
    )(t2d, bfc1r, Wfc2, bfc2r)


def _make_sc_conv(has_et):
    f32 = jnp.float32

    def body(*refs):
        if has_et:
            (asrc_h, adst_h, bound_h, h_h, srcr_h, dstr_h, etr_h,
             acc_out, den_out,
             asrc_v, adst_v, bound_v, srcb, etb, db0, db1, db2, db3,
             exrow, msub, h_sh, acc_sh, den_sh, sem) = refs
        else:
            (asrc_h, adst_h, bound_h, h_h, srcr_h, dstr_h,
             acc_out, den_out,
             asrc_v, adst_v, bound_v, srcb, db0, db1, db2, db3,
             exrow, msub, h_sh, acc_sh, den_sh, sem) = refs
            etb = None
        dstbs = (db0, db1, db2, db3)
        c = lax.axis_index("c")
        s = lax.axis_index("s")
        w = c * NS + s
        z16 = jnp.zeros((L,), f32)

        def zb(i, _):
            msub[i, pl.ds(0, L)] = z16
            msub[i, pl.ds(L, L)] = z16
            return 0
        lax.fori_loop(0, 128, zb, 0)
        for k in range(128 // L):
            exrow[pl.ds(k * L, L)] = z16
        pltpu.sync_copy(msub.at[pl.ds(0, 128)], acc_sh.at[pl.ds(s * 128, 128)])
        pltpu.sync_copy(exrow.at[pl.ds(0, 128)], den_sh.at[pl.ds(s * 128, 128)])
        pltpu.sync_copy(h_h.at[pl.ds(s * 128, 128)],
                        h_sh.at[pl.ds(s * 128, 128)])
        pltpu.sync_copy(asrc_h, asrc_v)
        pltpu.sync_copy(adst_h, adst_v)
        pltpu.sync_copy(bound_h, bound_v)
        plsc.subcore_barrier()

        for j in range(NSUB):
            row = w * NSUB + j
            dstb = dstbs[j]
            pltpu.sync_copy(srcr_h.at[row], srcb)
            pltpu.sync_copy(dstr_h.at[row], dstb)
            if has_et:
                pltpu.sync_copy(etr_h.at[row], etb)
            gat = pltpu.async_copy(h_sh.at[srcb], msub, sem)

            def grp(g, _):
                sl = pl.ds(g * L, L)
                s16 = srcb[sl]
                d16 = dstb[sl]
                vs = plsc.load_gather(asrc_v, [s16])
                vd = plsc.load_gather(adst_v, [d16])
                vb = plsc.load_gather(bound_v, [d16])
                al = vs + vd
                if has_et:
                    al = al + etb[sl]
                al = jnp.maximum(al, 0.2 * al)
                exrow[sl] = jnp.exp(al - vb)
                return 0
            lax.fori_loop(0, GPS, grp, 0)
            gat.wait()

            def scl(g, _):
                ex = exrow[pl.ds(g * L, L)]
                base = g * L
                for l in range(L):
                    e = ex[l]
                    r = base + l
                    msub[r, pl.ds(0, L)] = msub[r, pl.ds(0, L)] * e
                    msub[r, pl.ds(L, L)] = msub[r, pl.ds(L, L)] * e
                return 0
            lax.fori_loop(0, GPS, scl, 0)

            pltpu.sync_copy(msub, acc_sh.at[dstb], add=True)
            pltpu.sync_copy(exrow, den_sh.at[dstb], add=True)

        plsc.subcore_barrier()
        pltpu.sync_copy(acc_sh.at[pl.ds(s * 128, 128)],
                        acc_out.at[c, pl.ds(s * 128, 128)])
        pltpu.sync_copy(den_sh.at[pl.ds(s * 128, 128)],
                        den_out.at[c, pl.ds(s * 128, 128)])

    scratch = [
        pltpu.VMEM((N,), f32),
        pltpu.VMEM((N,), f32),
        pltpu.VMEM((N,), f32),
        pltpu.VMEM((SUB,), jnp.int32),
    ]
    if has_et:
        scratch.append(pltpu.VMEM((SUB,), f32))
    scratch += [
        pltpu.VMEM((SUB,), jnp.int32),
        pltpu.VMEM((SUB,), jnp.int32),
        pltpu.VMEM((SUB,), jnp.int32),
        pltpu.VMEM((SUB,), jnp.int32),
        pltpu.VMEM((SUB,), f32),
        pltpu.VMEM((SUB, DH), f32),
        pltpu.VMEM_SHARED((N, DH), f32),
        pltpu.VMEM_SHARED((N, DH), f32),
        pltpu.VMEM_SHARED((N,), f32),
        pltpu.SemaphoreType.DMA,
    ]
    return pl.kernel(
        body,
        out_type=[
            jax.ShapeDtypeStruct((NC, N, DH), f32),
            jax.ShapeDtypeStruct((NC, N), f32),
        ],
        mesh=_MESH,
        scratch_types=scratch,
        compiler_params=pltpu.CompilerParams(needs_layout_passes=False, use_tc_tiling_on_sc=False),
    )


_sc_conv1 = _make_sc_conv(True)
_sc_conv2 = _make_sc_conv(False)


def _sc3_body(ap_h, bt_h, srcr_h, dstr_h, wd_h, t_h,
              srcb, dstb, bufA, bufB, wdb, tbuf, tbx, sem):
    c = lax.axis_index("c")
    s = lax.axis_index("s")
    w = c * NS + s
    pltpu.sync_copy(wd_h, wdb)
    wv0 = wdb[pl.ds(0, L)]
    wv1 = wdb[pl.ds(L, L)]
    for j in range(NSUB):
        row = w * NSUB + j
        pltpu.sync_copy(srcr_h.at[row], srcb)
        pltpu.sync_copy(dstr_h.at[row], dstb)
        ca = pltpu.async_copy(ap_h.at[srcb], bufA, sem)
        ca.wait()
        cb = pltpu.async_copy(bt_h.at[dstb], bufB, sem)
        cb.wait()

        def grp(g, _):
            base = g * L
            for l in range(L):
                r = base + l
                a0 = bufA[r, pl.ds(0, L)]
                b0 = bufB[r, pl.ds(0, L)]
                a1 = bufA[r, pl.ds(L, L)]
                b1 = bufB[r, pl.ds(L, L)]
                sv = (jnp.maximum(a0 + b0, 0.0) * wv0
                      + jnp.maximum(a1 + b1, 0.0) * wv1)
                tbx[pl.ds(l * (L + 1), L)] = sv
            acc = jnp.zeros((L,), jnp.float32)
            rows17 = lax.iota(jnp.int32, L) * (L + 1)
            for cc in range(L):
                acc = acc + plsc.load_gather(tbx, [rows17 + cc])
            tbuf[pl.ds(base, L)] = acc
            return 0
        lax.fori_loop(0, GPS, grp, 0)
        pltpu.sync_copy(tbuf, t_h.at[pl.ds(w * EPW + j * SUB, SUB)])


_sc3 = pl.kernel(
    _sc3_body,
    out_type=jax.ShapeDtypeStruct((E,), jnp.float32),
    mesh=_MESH,
    scratch_types=[
        pltpu.VMEM((SUB,), jnp.int32),
        pltpu.VMEM((SUB,), jnp.int32),
        pltpu.VMEM((SUB, DH), jnp.float32),
        pltpu.VMEM((SUB, DH), jnp.float32),
        pltpu.VMEM((DH,), jnp.float32),
        pltpu.VMEM((SUB,), jnp.float32),
        pltpu.VMEM((L * (L + 1),), jnp.float32),
        pltpu.SemaphoreType.DMA,
    ],
    compiler_params=pltpu.CompilerParams(needs_layout_passes=False, use_tc_tiling_on_sc=False),
)


def kernel(x, edges, edge_feature, W1, att_src1, att_dst1, We1, att_e1, b1,
           W2, att_src2, att_dst2, b2, Wfc1, bfc1, Wfc2, bfc2):
    f32 = jnp.float32
    h1lin, asrc1, adst1, bound1, exself1, et2d = _tc1(
        x, W1, att_src1.reshape(DH, 1), att_dst1.reshape(DH, 1), We1.T,
        att_e1.reshape(1, DH), edge_feature.T)

    src_rows = edges[0].reshape(ROWS, SUB)
    dst_rows = edges[1].reshape(ROWS, SUB)
    et_rows = et2d.reshape(ROWS, SUB)

    acc1, den1 = _sc_conv1(asrc1, adst1, bound1, h1lin, src_rows, dst_rows,
                           et_rows)
    h2lin, asrc2, adst2, bound2, exself2 = _tc2(
        acc1.reshape(NC * N, DH), den1.reshape(NC * N, 1), exself1, h1lin,
        b1.reshape(1, DH), W2, att_src2.reshape(DH, 1),
        att_dst2.reshape(DH, 1))

    acc2, den2 = _sc_conv2(asrc2, adst2, bound2, h2lin, src_rows, dst_rows)
    ap, bt = _tc3(acc2.reshape(NC * N, DH), den2.reshape(NC * N, 1), exself2,
                  h2lin, b2.reshape(1, DH), Wfc1, bfc1.reshape(1, DH))

    wd = (Wfc2[:, 0] - Wfc2[:, 1]).astype(f32)
    tvec = _sc3(ap, bt, src_rows, dst_rows, wd)

    OUT = _tc4(tvec.reshape(512, 128), bfc1.reshape(1, DH), Wfc2,
               bfc2.reshape(1, 2))
    return OUT.transpose(0, 2, 1).reshape(N * N, 2)
